# baseline (speedup 1.0000x reference)
import numpy as np
import jax
import jax.numpy as jnp
from jax import lax
from jax.experimental import pallas as pl

_PRIOR_BOXES = np.array([[1.3221, 1.73145], [3.19275, 4.00944], [5.05587, 8.09892],
                         [9.47112, 4.84053], [11.2364, 10.0071]], dtype=np.float32) / 13.0
_NUM_PRIOR = 5
_NEL = 25
_IMG_W = 416.0
_IMG_H = 416.0
_LAMBDA_OBJ = 5.0
_LAMBDA_NONOBJ = 0.5


def _decode_body(dx, dy, p_ref, y_ref, o_ref):
    H, W = y_ref.shape[1], y_ref.shape[2]
    y0 = y_ref[0]
    mask = _LAMBDA_OBJ * y0 + _LAMBDA_NONOBJ * jnp.negative(y0 + (-1.0))
    gx = dx * lax.broadcasted_iota(jnp.int32, (H, W), 1).astype(jnp.float32)
    gy = dy * lax.broadcasted_iota(jnp.int32, (H, W), 0).astype(jnp.float32)
    for a in range(_NUM_PRIOR):
        s = a * _NEL
        pw = float(_PRIOR_BOXES[a, 0])
        ph = float(_PRIOR_BOXES[a, 1])
        o_ref[0, s] = p_ref[0, s] * mask
        t = jnp.trunc(dx * p_ref[0, s + 1])
        o_ref[0, s + 1] = (gx + t) * mask
        t = jnp.trunc(dy * p_ref[0, s + 2])
        o_ref[0, s + 2] = (gy + t) * mask
        t = jnp.trunc((pw * p_ref[0, s + 3]) * _IMG_W)
        o_ref[0, s + 3] = t * mask
        t = jnp.trunc((ph * p_ref[0, s + 4]) * _IMG_H)
        o_ref[0, s + 4] = t * mask
        o_ref[0, s + 5:s + _NEL] = p_ref[0, s + 5:s + _NEL] * mask[None]


def kernel(pred, y_hat):
    B, C, H, W = pred.shape
    grid_S = C
    dx = float(np.float32(_IMG_W / grid_S))
    dy = float(np.float32(_IMG_H / grid_S))

    y0 = y_hat[:, :, :, 0]

    import functools
    return pl.pallas_call(
        functools.partial(_decode_body, dx, dy),
        grid=(B,),
        in_specs=[
            pl.BlockSpec((1, C, H, W), lambda b: (b, 0, 0, 0)),
            pl.BlockSpec((1, H, W), lambda b: (b, 0, 0)),
        ],
        out_specs=pl.BlockSpec((1, C, H, W), lambda b: (b, 0, 0, 0)),
        out_shape=jax.ShapeDtypeStruct((B, C, H, W), jnp.float32),
    )(pred, y0)
